# Initial kernel scaffold; baseline (speedup 1.0000x reference)
import jax
import jax.numpy as jnp
from jax.experimental import pallas as pl


def kernel(edge_index1, edge_index2, W1, b1, W2, b2, W3, b3, W4, b4, W5, b5, W6, b6, W7, b7, W8, b8, W9, b9, Wm1, bm1, Wm2, bm2, Wm3, bm3):
    raise NotImplementedError("write your pallas kernel here")


# speedup vs baseline: 2.9711x; 2.9711x over previous
import functools

import jax
import jax.numpy as jnp
from jax import lax
from jax.experimental import pallas as pl
from jax.experimental.pallas import tpu as pltpu
from jax.experimental.pallas import tpu_sc as plsc

_N = 10000
_NP = 10240
_F = 128
_E = 320000
_NS = 16
_EPT = _E // _NS
_C = 80
_NCHUNK = _EPT // _C
_RPT = _NP // _NS
_ZR = 32


def _spmm_body(x_hbm, src_hbm, dst_hbm, out_hbm, acc, src_v, dst_v, rows_v,
               stage_v, sem):
  c = lax.axis_index("c")
  s = lax.axis_index("s")

  for r in range(_ZR):
    for j in range(_F // 16):
      stage_v[r, pl.ds(j * 16, 16)] = jnp.zeros((16,), jnp.float32)
  rbase = s * _RPT

  def zero_body(i, carry):
    pltpu.sync_copy(stage_v, acc.at[pl.ds(rbase + i * _ZR, _ZR)])
    return carry

  lax.fori_loop(0, _RPT // _ZR, zero_body, 0)
  plsc.subcore_barrier()

  ebase = c * _E + s * _EPT

  def edge_body(i, carry):
    off = ebase + i * _C
    pltpu.sync_copy(src_hbm.at[pl.ds(off, _C)], src_v)
    pltpu.sync_copy(dst_hbm.at[pl.ds(off, _C)], dst_v)
    pltpu.async_copy(x_hbm.at[src_v], rows_v, sem).wait()
    pltpu.sync_copy(rows_v, acc.at[dst_v], add=True)
    return carry

  lax.fori_loop(0, _NCHUNK, edge_body, 0)
  plsc.subcore_barrier()

  obase = c * _NP + rbase

  def wb_body(i, carry):
    pltpu.sync_copy(acc.at[pl.ds(rbase + i * _ZR, _ZR)], stage_v)
    pltpu.sync_copy(stage_v, out_hbm.at[pl.ds(obase + i * _ZR, _ZR)])
    return carry

  lax.fori_loop(0, _RPT // _ZR, wb_body, 0)


def _spmm(x, src, dst):
  mesh = plsc.VectorSubcoreMesh(core_axis_name="c", subcore_axis_name="s")
  k = pl.kernel(
      _spmm_body,
      out_type=jax.ShapeDtypeStruct((2 * _NP, _F), jnp.float32),
      mesh=mesh,
      scratch_types=[
          pltpu.VMEM_SHARED((_NP, _F), jnp.float32),
          pltpu.VMEM((_C,), jnp.int32),
          pltpu.VMEM((_C,), jnp.int32),
          pltpu.VMEM((_C, _F), jnp.float32),
          pltpu.VMEM((_ZR, _F), jnp.float32),
          pltpu.SemaphoreType.DMA,
      ],
  )
  return k(x, src, dst)


_R = 2048


def _layer_body(p_ref, b_ref, W_ref, sin_ref, Wm1_ref, bm1_ref, Wm2_ref,
                bm2_ref, wm3_ref, bm3_ref, h_ref, sout_ref):
  t = jnp.maximum(p_ref[...] + b_ref[...], 0.0)
  nrm = jnp.sqrt(jnp.sum(t * t, axis=1, keepdims=True))
  x = t / jnp.maximum(nrm, 1e-12)
  h_ref[...] = jnp.dot(x, W_ref[...], preferred_element_type=jnp.float32)
  h1 = jnp.maximum(
      jnp.dot(x, Wm1_ref[...], preferred_element_type=jnp.float32)
      + bm1_ref[...], 0.0)
  h2 = jnp.maximum(
      jnp.dot(h1, Wm2_ref[...], preferred_element_type=jnp.float32)
      + bm2_ref[...], 0.0)
  sc = jnp.sum(h2 * wm3_ref[...], axis=1, keepdims=True) + bm3_ref[0, 0]
  sout_ref[...] = sin_ref[...] + sc


def _layer(p, b, W, score, Wm1, bm1, Wm2, bm2, wm3, bm3):
  M = p.shape[0]
  grid = (M // _R,)
  full = lambda shape: pl.BlockSpec(shape, lambda i: (0, 0))
  h, sout = pl.pallas_call(
      _layer_body,
      grid=grid,
      in_specs=[
          pl.BlockSpec((_R, _F), lambda i: (i, 0)),
          full((1, _F)),
          full((_F, _F)),
          pl.BlockSpec((_R, 1), lambda i: (i, 0)),
          full((_F, _F // 2)),
          full((1, _F // 2)),
          full((_F // 2, _F // 4)),
          full((1, _F // 4)),
          full((1, _F // 4)),
          full((1, 1)),
      ],
      out_specs=[
          pl.BlockSpec((_R, _F), lambda i: (i, 0)),
          pl.BlockSpec((_R, 1), lambda i: (i, 0)),
      ],
      out_shape=[
          jax.ShapeDtypeStruct((M, _F), jnp.float32),
          jax.ShapeDtypeStruct((M, 1), jnp.float32),
      ],
  )(p, b, W, score, Wm1, bm1, Wm2, bm2, wm3, bm3)
  return h, sout


def _final_body(pa_ref, pb_ref, b_ref, sa_ref, sb_ref, Wm1_ref, bm1_ref,
                Wm2_ref, bm2_ref, wm3_ref, bm3_ref, out_ref):
  def mlp_score(p, sin):
    x = jnp.maximum(p + b_ref[...], 0.0)
    h1 = jnp.maximum(
        jnp.dot(x, Wm1_ref[...], preferred_element_type=jnp.float32)
        + bm1_ref[...], 0.0)
    h2 = jnp.maximum(
        jnp.dot(h1, Wm2_ref[...], preferred_element_type=jnp.float32)
        + bm2_ref[...], 0.0)
    return sin + jnp.sum(h2 * wm3_ref[...], axis=1, keepdims=True) + bm3_ref[0, 0]

  s1 = mlp_score(pa_ref[...], sa_ref[...])
  s2 = mlp_score(pb_ref[...], sb_ref[...])
  out_ref[...] = s1 * s2


_RF = 2000


def _final(pa, pb, b, sa, sb, Wm1, bm1, Wm2, bm2, wm3, bm3):
  grid = (_N // _RF,)
  full = lambda shape: pl.BlockSpec(shape, lambda i: (0, 0))
  return pl.pallas_call(
      _final_body,
      grid=grid,
      in_specs=[
          pl.BlockSpec((_RF, _F), lambda i: (i, 0)),
          pl.BlockSpec((_RF, _F), lambda i: (i, 0)),
          full((1, _F)),
          pl.BlockSpec((_RF, 1), lambda i: (i, 0)),
          pl.BlockSpec((_RF, 1), lambda i: (i, 0)),
          full((_F, _F // 2)),
          full((1, _F // 2)),
          full((_F // 2, _F // 4)),
          full((1, _F // 4)),
          full((1, _F // 4)),
          full((1, 1)),
      ],
      out_specs=pl.BlockSpec((_RF, 1), lambda i: (i, 0)),
      out_shape=jax.ShapeDtypeStruct((_N, 1), jnp.float32),
  )(pa, pb, b, sa, sb, Wm1, bm1, Wm2, bm2, wm3, bm3)


def kernel(edge_index1, edge_index2, W1, b1, W2, b2, W3, b3, W4, b4, W5, b5,
           W6, b6, W7, b7, W8, b8, W9, b9, Wm1, bm1, Wm2, bm2, Wm3, bm3):
  src_l1 = jnp.concatenate([edge_index1[0], edge_index2[0]])
  src_ln = jnp.concatenate([edge_index1[0], edge_index2[0] + _NP])
  dstcat = jnp.concatenate([edge_index1[1], edge_index2[1]])

  b1r = b1.reshape(1, _F)
  bm1r = bm1.reshape(1, _F // 2)
  bm2r = bm2.reshape(1, _F // 4)
  wm3r = Wm3.reshape(1, _F // 4)
  bm3r = bm3.reshape(1, 1)
  steps = [(b2, W3), (b3, W4), (b4, W5), (b5, W6), (b6, W7), (b7, W8),
           (b8, W9)]

  score = jnp.zeros((2 * _NP, 1), jnp.float32)
  p = _spmm(W1, src_l1, dstcat)
  x, score = _layer(p, b1r, W2, score, Wm1, bm1r, Wm2, bm2r, wm3r, bm3r)
  for (bi, Wn) in steps:
    p = _spmm(x, src_ln, dstcat)
    x, score = _layer(p, bi.reshape(1, _F), Wn, score, Wm1, bm1r, Wm2, bm2r,
                      wm3r, bm3r)
  p = _spmm(x, src_ln, dstcat)
  pa, pb = p[:_N], p[_NP:_NP + _N]
  sa, sb = score[:_N], score[_NP:_NP + _N]
  return _final(pa, pb, b9.reshape(1, _F), sa, sb, Wm1, bm1r, Wm2, bm2r,
                wm3r, bm3r)


# speedup vs baseline: 8.1028x; 2.7272x over previous
import functools

import jax
import jax.numpy as jnp
from jax import lax
from jax.experimental import pallas as pl
from jax.experimental.pallas import tpu as pltpu
from jax.experimental.pallas import tpu_sc as plsc

_N = 10000
_NP = 10240
_F = 128
_E = 320000
_NS = 16
_EPT = _E // _NS
_C = 50
_NCHUNK = _EPT // _C
_G = 40
_NGRP = _NCHUNK // _G
_NB = 5
_RPT = _NP // _NS
_ZR = 16


def _spmm_body(x_hbm, src_hbm, dst_hbm, out_hbm, acc, srcg, dstg, rows0,
               rows1, rows2, rows3, rows4, stage_v, g0, g1, g2, g3, g4):
  c = lax.axis_index("c")
  s = lax.axis_index("s")
  rows = [rows0, rows1, rows2, rows3, rows4]
  gsem = [g0, g1, g2, g3, g4]
  w = c * _NS + s

  for r in range(_ZR):
    for j in range(_F // 16):
      stage_v[r, pl.ds(j * 16, 16)] = jnp.zeros((16,), jnp.float32)
  rbase = s * _RPT

  def zero_body(i, carry):
    pltpu.sync_copy(stage_v, acc.at[pl.ds(rbase + i * _ZR, _ZR)])
    return carry

  lax.fori_loop(0, _RPT // _ZR, zero_body, 0)
  plsc.subcore_barrier()

  def grp_body(g, carry):
    pltpu.sync_copy(src_hbm.at[w * _NGRP + g], srcg)
    pltpu.sync_copy(dst_hbm.at[w * _NGRP + g], dstg)
    for b in range(_NB):
      pltpu.async_copy(x_hbm.at[srcg.at[b]], rows[b], gsem[b])

    def chunk_body(m, carry2):
      for b in range(_NB):
        l = m * _NB + b
        pltpu.make_async_copy(x_hbm.at[srcg.at[l]], rows[b], gsem[b]).wait()
        pltpu.sync_copy(rows[b], acc.at[dstg.at[l]], add=True)
        pltpu.async_copy(x_hbm.at[srcg.at[l + _NB]], rows[b], gsem[b])
      return carry2

    lax.fori_loop(0, _G // _NB - 1, chunk_body, 0)
    for b in range(_NB):
      l = _G - _NB + b
      pltpu.make_async_copy(x_hbm.at[srcg.at[l]], rows[b], gsem[b]).wait()
      pltpu.sync_copy(rows[b], acc.at[dstg.at[l]], add=True)
    return carry

  lax.fori_loop(0, _NGRP, grp_body, 0)
  plsc.subcore_barrier()

  obase = c * _NP + rbase

  def wb_body(i, carry):
    pltpu.sync_copy(acc.at[pl.ds(rbase + i * _ZR, _ZR)], stage_v)
    pltpu.sync_copy(stage_v, out_hbm.at[pl.ds(obase + i * _ZR, _ZR)])
    return carry

  lax.fori_loop(0, _RPT // _ZR, wb_body, 0)


def _spmm(x, src, dst):
  mesh = plsc.VectorSubcoreMesh(core_axis_name="c", subcore_axis_name="s")
  k = pl.kernel(
      _spmm_body,
      out_type=jax.ShapeDtypeStruct((2 * _NP, _F), jnp.float32),
      mesh=mesh,
      scratch_types=[
          pltpu.VMEM_SHARED((_NP, _F), jnp.float32),
          pltpu.VMEM((_G, _C), jnp.int32),
          pltpu.VMEM((_G, _C), jnp.int32),
      ] + [pltpu.VMEM((_C, _F), jnp.float32) for _ in range(_NB)] + [
          pltpu.VMEM((_ZR, _F), jnp.float32),
      ] + [pltpu.SemaphoreType.DMA for _ in range(_NB)],
  )
  return k(x, src.reshape(2 * _NS * _NGRP, _G, _C),
           dst.reshape(2 * _NS * _NGRP, _G, _C))


_R = 2048


def _layer_body(p_ref, b_ref, W_ref, sin_ref, Wm1_ref, bm1_ref, Wm2_ref,
                bm2_ref, wm3_ref, bm3_ref, h_ref, sout_ref):
  t = jnp.maximum(p_ref[...] + b_ref[...], 0.0)
  nrm = jnp.sqrt(jnp.sum(t * t, axis=1, keepdims=True))
  x = t / jnp.maximum(nrm, 1e-12)
  h_ref[...] = jnp.dot(x, W_ref[...], preferred_element_type=jnp.float32)
  h1 = jnp.maximum(
      jnp.dot(x, Wm1_ref[...], preferred_element_type=jnp.float32)
      + bm1_ref[...], 0.0)
  h2 = jnp.maximum(
      jnp.dot(h1, Wm2_ref[...], preferred_element_type=jnp.float32)
      + bm2_ref[...], 0.0)
  sc = jnp.sum(h2 * wm3_ref[...], axis=1, keepdims=True) + bm3_ref[0, 0]
  sout_ref[...] = sin_ref[...] + sc


def _layer(p, b, W, score, Wm1, bm1, Wm2, bm2, wm3, bm3):
  M = p.shape[0]
  grid = (M // _R,)
  full = lambda shape: pl.BlockSpec(shape, lambda i: (0, 0))
  h, sout = pl.pallas_call(
      _layer_body,
      grid=grid,
      in_specs=[
          pl.BlockSpec((_R, _F), lambda i: (i, 0)),
          full((1, _F)),
          full((_F, _F)),
          pl.BlockSpec((_R, 1), lambda i: (i, 0)),
          full((_F, _F // 2)),
          full((1, _F // 2)),
          full((_F // 2, _F // 4)),
          full((1, _F // 4)),
          full((1, _F // 4)),
          full((1, 1)),
      ],
      out_specs=[
          pl.BlockSpec((_R, _F), lambda i: (i, 0)),
          pl.BlockSpec((_R, 1), lambda i: (i, 0)),
      ],
      out_shape=[
          jax.ShapeDtypeStruct((M, _F), jnp.float32),
          jax.ShapeDtypeStruct((M, 1), jnp.float32),
      ],
  )(p, b, W, score, Wm1, bm1, Wm2, bm2, wm3, bm3)
  return h, sout


def _final_body(pa_ref, pb_ref, b_ref, sa_ref, sb_ref, Wm1_ref, bm1_ref,
                Wm2_ref, bm2_ref, wm3_ref, bm3_ref, out_ref):
  def mlp_score(p, sin):
    x = jnp.maximum(p + b_ref[...], 0.0)
    h1 = jnp.maximum(
        jnp.dot(x, Wm1_ref[...], preferred_element_type=jnp.float32)
        + bm1_ref[...], 0.0)
    h2 = jnp.maximum(
        jnp.dot(h1, Wm2_ref[...], preferred_element_type=jnp.float32)
        + bm2_ref[...], 0.0)
    return sin + jnp.sum(h2 * wm3_ref[...], axis=1, keepdims=True) + bm3_ref[0, 0]

  s1 = mlp_score(pa_ref[...], sa_ref[...])
  s2 = mlp_score(pb_ref[...], sb_ref[...])
  out_ref[...] = s1 * s2


_RF = 2000


def _final(pa, pb, b, sa, sb, Wm1, bm1, Wm2, bm2, wm3, bm3):
  grid = (_N // _RF,)
  full = lambda shape: pl.BlockSpec(shape, lambda i: (0, 0))
  return pl.pallas_call(
      _final_body,
      grid=grid,
      in_specs=[
          pl.BlockSpec((_RF, _F), lambda i: (i, 0)),
          pl.BlockSpec((_RF, _F), lambda i: (i, 0)),
          full((1, _F)),
          pl.BlockSpec((_RF, 1), lambda i: (i, 0)),
          pl.BlockSpec((_RF, 1), lambda i: (i, 0)),
          full((_F, _F // 2)),
          full((1, _F // 2)),
          full((_F // 2, _F // 4)),
          full((1, _F // 4)),
          full((1, _F // 4)),
          full((1, 1)),
      ],
      out_specs=pl.BlockSpec((_RF, 1), lambda i: (i, 0)),
      out_shape=jax.ShapeDtypeStruct((_N, 1), jnp.float32),
  )(pa, pb, b, sa, sb, Wm1, bm1, Wm2, bm2, wm3, bm3)


def kernel(edge_index1, edge_index2, W1, b1, W2, b2, W3, b3, W4, b4, W5, b5,
           W6, b6, W7, b7, W8, b8, W9, b9, Wm1, bm1, Wm2, bm2, Wm3, bm3):
  src_l1 = jnp.concatenate([edge_index1[0], edge_index2[0]])
  src_ln = jnp.concatenate([edge_index1[0], edge_index2[0] + _NP])
  dstcat = jnp.concatenate([edge_index1[1], edge_index2[1]])

  b1r = b1.reshape(1, _F)
  bm1r = bm1.reshape(1, _F // 2)
  bm2r = bm2.reshape(1, _F // 4)
  wm3r = Wm3.reshape(1, _F // 4)
  bm3r = bm3.reshape(1, 1)
  steps = [(b2, W3), (b3, W4), (b4, W5), (b5, W6), (b6, W7), (b7, W8),
           (b8, W9)]

  score = jnp.zeros((2 * _NP, 1), jnp.float32)
  p = _spmm(W1, src_l1, dstcat)
  x, score = _layer(p, b1r, W2, score, Wm1, bm1r, Wm2, bm2r, wm3r, bm3r)
  for (bi, Wn) in steps:
    p = _spmm(x, src_ln, dstcat)
    x, score = _layer(p, bi.reshape(1, _F), Wn, score, Wm1, bm1r, Wm2, bm2r,
                      wm3r, bm3r)
  p = _spmm(x, src_ln, dstcat)
  pa, pb = p[:_N], p[_NP:_NP + _N]
  sa, sb = score[:_N], score[_NP:_NP + _N]
  return _final(pa, pb, b9.reshape(1, _F), sa, sb, Wm1, bm1r, Wm2, bm2r,
                wm3r, bm3r)


# speedup vs baseline: 8.1514x; 1.0060x over previous
import functools

import jax
import jax.numpy as jnp
from jax import lax
from jax.experimental import pallas as pl
from jax.experimental.pallas import tpu as pltpu
from jax.experimental.pallas import tpu_sc as plsc

_N = 10000
_NP = 10240
_F = 128
_E = 320000
_NS = 16
_EPT = _E // _NS
_C = 50
_NCHUNK = _EPT // _C
_G = 40
_NGRP = _NCHUNK // _G
_NB = 5
_RPT = _NP // _NS
_ZR = 16


def _spmm_body(x_hbm, src_hbm, dst_hbm, out_hbm, acc, srcg, dstg, rows0,
               rows1, rows2, rows3, rows4, stage_v, g0, g1, g2, g3, g4, ssem,
               zsem):
  c = lax.axis_index("c")
  s = lax.axis_index("s")
  rows = [rows0, rows1, rows2, rows3, rows4]
  gsem = [g0, g1, g2, g3, g4]
  w = c * _NS + s
  rbase = s * _RPT

  def swait():
    pltpu.make_async_copy(rows[0], acc.at[dstg.at[0]], ssem).wait()

  for r in range(_ZR):
    for j in range(_F // 16):
      stage_v[r, pl.ds(j * 16, 16)] = jnp.zeros((16,), jnp.float32)

  def zissue(i, carry):
    pltpu.async_copy(stage_v, acc.at[pl.ds(rbase + i * _ZR, _ZR)], zsem)
    return carry

  lax.fori_loop(0, _RPT // _ZR, zissue, 0)

  def zdrain(i, carry):
    pltpu.make_async_copy(stage_v, acc.at[pl.ds(rbase, _ZR)], zsem).wait()
    return carry

  lax.fori_loop(0, _RPT // _ZR, zdrain, 0)
  plsc.subcore_barrier()

  def grp_body(g, carry):
    pltpu.sync_copy(src_hbm.at[w * _NGRP + g], srcg)
    pltpu.sync_copy(dst_hbm.at[w * _NGRP + g], dstg)
    for b in range(_NB - 1):
      pltpu.async_copy(x_hbm.at[srcg.at[b]], rows[b], gsem[b])

    def chunk_body(m, carry2):
      for b in range(_NB):
        l = m * _NB + b
        bn = (b + _NB - 1) % _NB
        pltpu.make_async_copy(x_hbm.at[srcg.at[l]], rows[b], gsem[b]).wait()
        pltpu.sync_copy(rows[b], acc.at[dstg.at[l]], add=True)

        def _issue(_l=l, _bn=bn):
          pltpu.async_copy(x_hbm.at[srcg.at[_l + _NB - 1]], rows[_bn],
                           gsem[_bn])

        if b == 0:
          _issue()
        else:
          pl.when(m < _G // _NB - 1)(_issue)
      return carry2

    lax.fori_loop(0, _G // _NB, chunk_body, 0)
    return carry

  lax.fori_loop(0, _NGRP, grp_body, 0)
  plsc.subcore_barrier()

  obase = c * _NP + rbase

  def wb_body(i, carry):
    pltpu.sync_copy(acc.at[pl.ds(rbase + i * _ZR, _ZR)], stage_v)
    pltpu.sync_copy(stage_v, out_hbm.at[pl.ds(obase + i * _ZR, _ZR)])
    return carry

  lax.fori_loop(0, _RPT // _ZR, wb_body, 0)


def _spmm(x, src, dst):
  mesh = plsc.VectorSubcoreMesh(core_axis_name="c", subcore_axis_name="s")
  k = pl.kernel(
      _spmm_body,
      out_type=jax.ShapeDtypeStruct((2 * _NP, _F), jnp.float32),
      mesh=mesh,
      scratch_types=[
          pltpu.VMEM_SHARED((_NP, _F), jnp.float32),
          pltpu.VMEM((_G, _C), jnp.int32),
          pltpu.VMEM((_G, _C), jnp.int32),
      ] + [pltpu.VMEM((_C, _F), jnp.float32) for _ in range(_NB)] + [
          pltpu.VMEM((_ZR, _F), jnp.float32),
      ] + [pltpu.SemaphoreType.DMA for _ in range(_NB + 2)],
  )
  return k(x, src.reshape(2 * _NS * _NGRP, _G, _C),
           dst.reshape(2 * _NS * _NGRP, _G, _C))


_R = 2048


def _layer_body(p_ref, b_ref, W_ref, sin_ref, Wm1_ref, bm1_ref, Wm2_ref,
                bm2_ref, wm3_ref, bm3_ref, h_ref, sout_ref):
  t = jnp.maximum(p_ref[...] + b_ref[...], 0.0)
  nrm = jnp.sqrt(jnp.sum(t * t, axis=1, keepdims=True))
  x = t / jnp.maximum(nrm, 1e-12)
  h_ref[...] = jnp.dot(x, W_ref[...], preferred_element_type=jnp.float32)
  h1 = jnp.maximum(
      jnp.dot(x, Wm1_ref[...], preferred_element_type=jnp.float32)
      + bm1_ref[...], 0.0)
  h2 = jnp.maximum(
      jnp.dot(h1, Wm2_ref[...], preferred_element_type=jnp.float32)
      + bm2_ref[...], 0.0)
  sc = jnp.sum(h2 * wm3_ref[...], axis=1, keepdims=True) + bm3_ref[0, 0]
  sout_ref[...] = sin_ref[...] + sc


def _layer(p, b, W, score, Wm1, bm1, Wm2, bm2, wm3, bm3):
  M = p.shape[0]
  grid = (M // _R,)
  full = lambda shape: pl.BlockSpec(shape, lambda i: (0, 0))
  h, sout = pl.pallas_call(
      _layer_body,
      grid=grid,
      in_specs=[
          pl.BlockSpec((_R, _F), lambda i: (i, 0)),
          full((1, _F)),
          full((_F, _F)),
          pl.BlockSpec((_R, 1), lambda i: (i, 0)),
          full((_F, _F // 2)),
          full((1, _F // 2)),
          full((_F // 2, _F // 4)),
          full((1, _F // 4)),
          full((1, _F // 4)),
          full((1, 1)),
      ],
      out_specs=[
          pl.BlockSpec((_R, _F), lambda i: (i, 0)),
          pl.BlockSpec((_R, 1), lambda i: (i, 0)),
      ],
      out_shape=[
          jax.ShapeDtypeStruct((M, _F), jnp.float32),
          jax.ShapeDtypeStruct((M, 1), jnp.float32),
      ],
  )(p, b, W, score, Wm1, bm1, Wm2, bm2, wm3, bm3)
  return h, sout


def _final_body(pa_ref, pb_ref, b_ref, sa_ref, sb_ref, Wm1_ref, bm1_ref,
                Wm2_ref, bm2_ref, wm3_ref, bm3_ref, out_ref):
  def mlp_score(p, sin):
    x = jnp.maximum(p + b_ref[...], 0.0)
    h1 = jnp.maximum(
        jnp.dot(x, Wm1_ref[...], preferred_element_type=jnp.float32)
        + bm1_ref[...], 0.0)
    h2 = jnp.maximum(
        jnp.dot(h1, Wm2_ref[...], preferred_element_type=jnp.float32)
        + bm2_ref[...], 0.0)
    return sin + jnp.sum(h2 * wm3_ref[...], axis=1, keepdims=True) + bm3_ref[0, 0]

  s1 = mlp_score(pa_ref[...], sa_ref[...])
  s2 = mlp_score(pb_ref[...], sb_ref[...])
  out_ref[...] = s1 * s2


_RF = 2000


def _final(pa, pb, b, sa, sb, Wm1, bm1, Wm2, bm2, wm3, bm3):
  grid = (_N // _RF,)
  full = lambda shape: pl.BlockSpec(shape, lambda i: (0, 0))
  return pl.pallas_call(
      _final_body,
      grid=grid,
      in_specs=[
          pl.BlockSpec((_RF, _F), lambda i: (i, 0)),
          pl.BlockSpec((_RF, _F), lambda i: (i, 0)),
          full((1, _F)),
          pl.BlockSpec((_RF, 1), lambda i: (i, 0)),
          pl.BlockSpec((_RF, 1), lambda i: (i, 0)),
          full((_F, _F // 2)),
          full((1, _F // 2)),
          full((_F // 2, _F // 4)),
          full((1, _F // 4)),
          full((1, _F // 4)),
          full((1, 1)),
      ],
      out_specs=pl.BlockSpec((_RF, 1), lambda i: (i, 0)),
      out_shape=jax.ShapeDtypeStruct((_N, 1), jnp.float32),
  )(pa, pb, b, sa, sb, Wm1, bm1, Wm2, bm2, wm3, bm3)


def kernel(edge_index1, edge_index2, W1, b1, W2, b2, W3, b3, W4, b4, W5, b5,
           W6, b6, W7, b7, W8, b8, W9, b9, Wm1, bm1, Wm2, bm2, Wm3, bm3):
  src_l1 = jnp.concatenate([edge_index1[0], edge_index2[0]])
  src_ln = jnp.concatenate([edge_index1[0], edge_index2[0] + _NP])
  dstcat = jnp.concatenate([edge_index1[1], edge_index2[1]])

  b1r = b1.reshape(1, _F)
  bm1r = bm1.reshape(1, _F // 2)
  bm2r = bm2.reshape(1, _F // 4)
  wm3r = Wm3.reshape(1, _F // 4)
  bm3r = bm3.reshape(1, 1)
  steps = [(b2, W3), (b3, W4), (b4, W5), (b5, W6), (b6, W7), (b7, W8),
           (b8, W9)]

  score = jnp.zeros((2 * _NP, 1), jnp.float32)
  p = _spmm(W1, src_l1, dstcat)
  x, score = _layer(p, b1r, W2, score, Wm1, bm1r, Wm2, bm2r, wm3r, bm3r)
  for (bi, Wn) in steps:
    p = _spmm(x, src_ln, dstcat)
    x, score = _layer(p, bi.reshape(1, _F), Wn, score, Wm1, bm1r, Wm2, bm2r,
                      wm3r, bm3r)
  p = _spmm(x, src_ln, dstcat)
  pa, pb = p[:_N], p[_NP:_NP + _N]
  sa, sb = score[:_N], score[_NP:_NP + _N]
  return _final(pa, pb, b9.reshape(1, _F), sa, sb, Wm1, bm1r, Wm2, bm2r,
                wm3r, bm3r)
